# baseline (speedup 1.0000x reference)
import jax
import jax.numpy as jnp
from jax import lax
from jax.experimental import pallas as pl
from jax.experimental.pallas import tpu as pltpu
from jax.experimental.pallas import tpu_sc as plsc

BATCH = 4096
HIST = 50
DIM = 128

NC = 2
NS = 16
NW = NC * NS

TOTAL = BATCH * HIST
CHUNK = 128
N_CHUNKS = HIST
NBUF = 5
LEAD = 3


def _emb_kernel(idx_hbm, table_hbm, out_hbm, idx_v, rows_v, gsems, osems):
    wid = lax.axis_index("s") * NC + lax.axis_index("c")
    base = wid * CHUNK

    pltpu.sync_copy(idx_hbm.at[pl.ds(0, HIST), pl.ds(wid * CHUNK, CHUNK)],
                    idx_v)

    for b in range(LEAD):
        pltpu.async_copy(table_hbm.at[idx_v.at[b]], rows_v.at[b], gsems.at[b])

    @pl.loop(0, N_CHUNKS, step=NBUF)
    def _group(g):
        for b in range(NBUF):
            j = g + b
            pltpu.make_async_copy(table_hbm.at[idx_v.at[0]], rows_v.at[b],
                                  gsems.at[b]).wait()
            pltpu.async_copy(rows_v.at[b],
                             out_hbm.at[pl.ds(j * BATCH + base, CHUNK)],
                             osems.at[b])

            bb = (b + LEAD) % NBUF

            @pl.when(j + LEAD < N_CHUNKS)
            def _():
                @pl.when(j + LEAD >= NBUF)
                def _():
                    pltpu.make_async_copy(rows_v.at[bb],
                                          out_hbm.at[pl.ds(0, CHUNK)],
                                          osems.at[bb]).wait()

                pltpu.async_copy(table_hbm.at[idx_v.at[j + LEAD]],
                                 rows_v.at[bb], gsems.at[bb])

    for b in range(NBUF):
        pltpu.make_async_copy(rows_v.at[b], out_hbm.at[pl.ds(0, CHUNK)],
                              osems.at[b]).wait()


@jax.jit
def kernel(token_ids, weight):
    idx = token_ids.astype(jnp.int32).T
    mesh = plsc.VectorSubcoreMesh(core_axis_name="c", subcore_axis_name="s",
                                  num_cores=NC, num_subcores=NS)
    out = pl.kernel(
        _emb_kernel,
        out_type=jax.ShapeDtypeStruct((TOTAL, DIM), jnp.float32),
        mesh=mesh,
        scratch_types=[
            pltpu.VMEM((N_CHUNKS, CHUNK), jnp.int32),
            pltpu.VMEM((NBUF, CHUNK, DIM), jnp.float32),
            pltpu.SemaphoreType.DMA((NBUF,)),
            pltpu.SemaphoreType.DMA((NBUF,)),
        ],
    )(idx, weight)
    return out.reshape(HIST, BATCH, DIM).transpose(1, 0, 2)
